# baseline (speedup 1.0000x reference)
import functools

import jax
import jax.numpy as jnp
from jax import lax
from jax.experimental import pallas as pl
from jax.experimental.pallas import tpu as pltpu
from jax.experimental.pallas import tpu_sc as plsc

BATCH = 16384
D = 32
NW = 32
BPW = BATCH // NW
CH = 128
NCH = BPW // CH
L = 16


def _gmf_body(items_r, users_r, items_embT_r, users_embT_r, out_r,
              it_idx, us_idx, a_cols, b_cols, out_v, sem):
    wid = lax.axis_index("s") * 2 + lax.axis_index("c")
    base = wid * BPW

    cp1 = pltpu.async_copy(items_r.at[pl.ds(base, BPW)], it_idx, sem)
    cp2 = pltpu.async_copy(users_r.at[pl.ds(base, BPW)], us_idx, sem)
    cp1.wait()
    cp2.wait()

    def fire(d, carry):
        for ch in range(NCH):
            sl = pl.ds(ch * CH, CH)
            pltpu.async_copy(items_embT_r.at[d].at[it_idx.at[sl]],
                             a_cols.at[d].at[sl], sem)
            pltpu.async_copy(users_embT_r.at[d].at[us_idx.at[sl]],
                             b_cols.at[d].at[sl], sem)
        return carry

    lax.fori_loop(0, D, fire, 0)

    pltpu.make_async_copy(items_embT_r.at[pl.ds(0, D), pl.ds(0, BPW)],
                          a_cols, sem).wait()
    pltpu.make_async_copy(users_embT_r.at[pl.ds(0, D), pl.ds(0, BPW)],
                          b_cols, sem).wait()

    lane = lax.iota(jnp.int32, L)

    def group(g, carry):
        acc = jnp.zeros((L,), jnp.float32)
        for d in range(D):
            pa = a_cols[d, pl.ds(g * L, L)]
            pb = b_cols[d, pl.ds(g * L, L)]
            acc = acc + pa * pb
        sig = 1.0 / (1.0 + jnp.exp(-acc))
        out_v[pl.ds(g * L, L)] = sig
        return carry

    lax.fori_loop(0, BPW // L, group, 0)

    pltpu.sync_copy(out_v, out_r.at[pl.ds(base, BPW)])


@jax.jit
def _gmf(items, users, items_embedding, users_embedding):
    mesh = plsc.VectorSubcoreMesh(core_axis_name="c", subcore_axis_name="s")
    kfn = functools.partial(
        pl.kernel,
        mesh=mesh,
        out_type=jax.ShapeDtypeStruct((BATCH,), jnp.float32),
        scratch_types=[
            pltpu.VMEM((BPW,), jnp.int32),
            pltpu.VMEM((BPW,), jnp.int32),
            pltpu.VMEM((D, BPW), jnp.float32),
            pltpu.VMEM((D, BPW), jnp.float32),
            pltpu.VMEM((BPW,), jnp.float32),
            pltpu.SemaphoreType.DMA,
        ],
        compiler_params=pltpu.CompilerParams(
            needs_layout_passes=False, use_tc_tiling_on_sc=False),
    )(_gmf_body)
    return kfn(items, users, items_embedding.T, users_embedding.T)


def kernel(items, users, items_embedding, users_embedding):
    return _gmf(items.astype(jnp.int32), users.astype(jnp.int32),
                items_embedding, users_embedding)


# speedup vs baseline: 19.8588x; 19.8588x over previous
import functools

import jax
import jax.numpy as jnp
from jax import lax
from jax.experimental import pallas as pl
from jax.experimental.pallas import tpu as pltpu
from jax.experimental.pallas import tpu_sc as plsc

BATCH = 16384
D = 32
NW = 32
BPW = BATCH // NW
NB = 8
L = 16


def _gmf_body(items_r, users_r, items_embT_r, users_embT_r, out_r,
              it_idx, us_idx, a_blks, b_blks, sums_v, out_v, sem):
    wid = lax.axis_index("s") * 2 + lax.axis_index("c")
    base = wid * BPW

    cp1 = pltpu.async_copy(items_r.at[pl.ds(base, BPW)], it_idx, sem)
    cp2 = pltpu.async_copy(users_r.at[pl.ds(base, BPW)], us_idx, sem)
    cp1.wait()
    cp2.wait()

    lane = lax.iota(jnp.int32, L)
    lane_hi = lane + L
    lane0 = lane == 0

    def group(g, carry):
        va = it_idx[pl.ds(g * L, L)]
        vb = us_idx[pl.ds(g * L, L)]
        for half in range(2):
            for j in range(NB):
                jj = half * NB + j
                ra = va[jj]
                rb = vb[jj]
                sa = pl.multiple_of(ra - (ra & (L * 8 - 1)), L * 8)
                sb = pl.multiple_of(rb - (rb & (L * 8 - 1)), L * 8)
                pltpu.async_copy(
                    items_embT_r.at[pl.ds(0, D), pl.ds(sa, L * 8)],
                    a_blks.at[j], sem)
                pltpu.async_copy(
                    users_embT_r.at[pl.ds(0, D), pl.ds(sb, L * 8)],
                    b_blks.at[j], sem)
            for j in range(NB):
                pltpu.make_async_copy(
                    items_embT_r.at[pl.ds(0, D), pl.ds(0, L * 8)],
                    a_blks.at[j], sem).wait()
                pltpu.make_async_copy(
                    users_embT_r.at[pl.ds(0, D), pl.ds(0, L * 8)],
                    b_blks.at[j], sem).wait()
            for j in range(NB):
                jj = half * NB + j
                la = jnp.broadcast_to(va[jj] & (L * 8 - 1), (L,))
                lb = jnp.broadcast_to(vb[jj] & (L * 8 - 1), (L,))
                cj = jnp.broadcast_to(jnp.int32(j), (L,))
                a0 = plsc.load_gather(a_blks, [cj, lane, la])
                a1 = plsc.load_gather(a_blks, [cj, lane_hi, la])
                b0 = plsc.load_gather(b_blks, [cj, lane, lb])
                b1 = plsc.load_gather(b_blks, [cj, lane_hi, lb])
                p = a0 * b0 + a1 * b1
                s = jnp.sum(p)
                plsc.store_scatter(
                    sums_v, [jnp.broadcast_to(g * L + jj, (L,))],
                    jnp.broadcast_to(s, (L,)), mask=lane0)
        return carry

    lax.fori_loop(0, BPW // L, group, 0)

    def sig_pass(k, carry):
        v = sums_v[pl.ds(k * L, L)]
        out_v[pl.ds(k * L, L)] = 1.0 / (1.0 + jnp.exp(-v))
        return carry

    lax.fori_loop(0, BPW // L, sig_pass, 0)

    pltpu.sync_copy(out_v, out_r.at[pl.ds(base, BPW)])


@jax.jit
def _gmf(items, users, items_embedding, users_embedding):
    mesh = plsc.VectorSubcoreMesh(core_axis_name="c", subcore_axis_name="s")
    kfn = functools.partial(
        pl.kernel,
        mesh=mesh,
        out_type=jax.ShapeDtypeStruct((BATCH,), jnp.float32),
        scratch_types=[
            pltpu.VMEM((BPW,), jnp.int32),
            pltpu.VMEM((BPW,), jnp.int32),
            pltpu.VMEM((NB, D, L * 8), jnp.float32),
            pltpu.VMEM((NB, D, L * 8), jnp.float32),
            pltpu.VMEM((BPW,), jnp.float32),
            pltpu.VMEM((BPW,), jnp.float32),
            pltpu.SemaphoreType.DMA,
        ],
        compiler_params=pltpu.CompilerParams(needs_layout_passes=False),
    )(_gmf_body)
    return kfn(items, users, items_embedding.T, users_embedding.T)


def kernel(items, users, items_embedding, users_embedding):
    return _gmf(items.astype(jnp.int32), users.astype(jnp.int32),
                items_embedding, users_embedding)
